# baseline (speedup 1.0000x reference)
import jax
import jax.numpy as jnp
from jax.experimental import pallas as pl
from jax.experimental.pallas import tpu as pltpu

N, M, E, D = 16, 512, 768, 1024


def _main_body(lens_ref, x_ref, w_ref, b_ref, q_in_ref, wq_ref, bq_ref,
               p_ref, lens_v_ref, out2_ref, out1_ref, qt_scratch, dots_s):
    n = pl.program_id(0)

    @pl.when(n == 0)
    def _():
        qz = jnp.dot(q_in_ref[...], wq_ref[...],
                     preferred_element_type=jnp.float32) + bq_ref[...]
        q = jnp.tanh(qz)
        qt_scratch[...] = q.T.astype(jnp.bfloat16)

    L = lens_ref[n]

    row_ids = jax.lax.broadcasted_iota(jnp.int32, (M, 1), 0)
    mask = (row_ids < L).astype(jnp.float32)
    xm = x_ref[0] * mask
    out2_ref[0] = xm
    z = jnp.dot(xm.astype(jnp.bfloat16), w_ref[...],
                preferred_element_type=jnp.float32) + b_ref[...]
    t = jnp.tanh(z)
    d = jnp.dot(t.astype(jnp.bfloat16), qt_scratch[...],
                preferred_element_type=jnp.float32)

    lane_ids = jax.lax.broadcasted_iota(jnp.int32, (M, N), 1)
    dots_s[...] = jnp.where(lane_ids == n, jnp.broadcast_to(d, (M, N)),
                            dots_s[...])

    @pl.when(n == N - 1)
    def _():
        dd = dots_s[...]
        m1 = jnp.max(dd, axis=0, keepdims=True)
        e1 = jnp.exp(dd - m1)
        a = e1 / jnp.sum(e1, axis=0, keepdims=True)
        logits = a * (p_ref[...] * 10.0)
        g = jnp.max(logits)
        e2 = jnp.exp(logits - g)
        w = e2 / jnp.sum(e2)
        w = jnp.where(row_ids < lens_v_ref[...], w, 0.0)
        out1_ref[...] = w.T


@jax.jit
def kernel(query, groups, probabilities, group_lens, W_nodes, b_nodes,
           W_query, b_query):
    b_nodes2 = b_nodes.reshape(1, D)
    b_query2 = b_query.reshape(1, D)
    lens_row = group_lens.reshape(1, N)
    w_nodes_bf = W_nodes.astype(jnp.bfloat16)
    query_bf = query.astype(jnp.bfloat16)
    wq_bf = W_query.astype(jnp.bfloat16)

    grid_spec = pltpu.PrefetchScalarGridSpec(
        num_scalar_prefetch=1,
        grid=(N,),
        in_specs=[
            pl.BlockSpec((1, M, E), lambda n, lens: (n, 0, 0)),
            pl.BlockSpec((E, D), lambda n, lens: (0, 0)),
            pl.BlockSpec((1, D), lambda n, lens: (0, 0)),
            pl.BlockSpec((1, D), lambda n, lens: (0, 0)),
            pl.BlockSpec((D, D), lambda n, lens: (0, 0)),
            pl.BlockSpec((1, D), lambda n, lens: (0, 0)),
            pl.BlockSpec((1, N), lambda n, lens: (0, 0)),
            pl.BlockSpec((1, N), lambda n, lens: (0, 0)),
        ],
        out_specs=[
            pl.BlockSpec((1, M, E), lambda n, lens: (n, 0, 0)),
            pl.BlockSpec((N, M), lambda n, lens: (0, 0)),
        ],
        scratch_shapes=[
            pltpu.VMEM((D, 1), jnp.bfloat16),
            pltpu.VMEM((M, N), jnp.float32),
        ],
    )
    out2, w = pl.pallas_call(
        _main_body,
        grid_spec=grid_spec,
        out_shape=[
            jax.ShapeDtypeStruct((N, M, E), jnp.float32),
            jax.ShapeDtypeStruct((N, M), jnp.float32),
        ],
    )(group_lens, groups, w_nodes_bf, b_nodes2, query_bf, wq_bf, b_query2,
      probabilities, lens_row)

    return (w.reshape(N, M, 1), out2)


# speedup vs baseline: 1.1365x; 1.1365x over previous
import jax
import jax.numpy as jnp
from jax.experimental import pallas as pl
from jax.experimental.pallas import tpu as pltpu

N, M, E, D = 16, 512, 768, 1024


def _main_body(lens_ref, x_ref, w_ref, b_ref, q_in_ref, wq_ref, bq_ref,
               p_ref, lens_v_ref, out2_ref, out1_ref, qt_scratch, dots_s,
               w_bf):
    n = pl.program_id(0)

    @pl.when(n == 0)
    def _():
        qz = jnp.dot(q_in_ref[...].astype(jnp.bfloat16),
                     wq_ref[...].astype(jnp.bfloat16),
                     preferred_element_type=jnp.float32) + bq_ref[...]
        q = jnp.tanh(qz)
        qt_scratch[...] = q.T.astype(jnp.bfloat16)
        w_bf[...] = w_ref[...].astype(jnp.bfloat16)

    L = lens_ref[n]

    row_ids = jax.lax.broadcasted_iota(jnp.int32, (M, 1), 0)
    mask = (row_ids < L).astype(jnp.float32)
    xm = x_ref[0] * mask
    out2_ref[0] = xm
    z = jnp.dot(xm.astype(jnp.bfloat16), w_bf[...],
                preferred_element_type=jnp.float32) + b_ref[...]
    t = jnp.tanh(z)
    d = jnp.dot(t.astype(jnp.bfloat16), qt_scratch[...],
                preferred_element_type=jnp.float32)

    lane_ids = jax.lax.broadcasted_iota(jnp.int32, (M, N), 1)
    dots_s[...] = jnp.where(lane_ids == n, jnp.broadcast_to(d, (M, N)),
                            dots_s[...])

    @pl.when(n == N - 1)
    def _():
        dd = dots_s[...]
        m1 = jnp.max(dd, axis=0, keepdims=True)
        e1 = jnp.exp(dd - m1)
        a = e1 / jnp.sum(e1, axis=0, keepdims=True)
        logits = a * (p_ref[...] * 10.0)
        g = jnp.max(logits)
        e2 = jnp.exp(logits - g)
        w = e2 / jnp.sum(e2)
        w = jnp.where(row_ids < lens_v_ref[...], w, 0.0)
        out1_ref[...] = w.T


@jax.jit
def kernel(query, groups, probabilities, group_lens, W_nodes, b_nodes,
           W_query, b_query):
    b_nodes2 = b_nodes.reshape(1, D)
    b_query2 = b_query.reshape(1, D)
    lens_row = group_lens.reshape(1, N)

    grid_spec = pltpu.PrefetchScalarGridSpec(
        num_scalar_prefetch=1,
        grid=(N,),
        in_specs=[
            pl.BlockSpec((1, M, E), lambda n, lens: (n, 0, 0)),
            pl.BlockSpec((E, D), lambda n, lens: (0, 0)),
            pl.BlockSpec((1, D), lambda n, lens: (0, 0)),
            pl.BlockSpec((1, D), lambda n, lens: (0, 0)),
            pl.BlockSpec((D, D), lambda n, lens: (0, 0)),
            pl.BlockSpec((1, D), lambda n, lens: (0, 0)),
            pl.BlockSpec((1, N), lambda n, lens: (0, 0)),
            pl.BlockSpec((1, N), lambda n, lens: (0, 0)),
        ],
        out_specs=[
            pl.BlockSpec((1, M, E), lambda n, lens: (n, 0, 0)),
            pl.BlockSpec((N, M), lambda n, lens: (0, 0)),
        ],
        scratch_shapes=[
            pltpu.VMEM((D, 1), jnp.bfloat16),
            pltpu.VMEM((M, N), jnp.float32),
            pltpu.VMEM((E, D), jnp.bfloat16),
        ],
    )
    out2, w = pl.pallas_call(
        _main_body,
        grid_spec=grid_spec,
        out_shape=[
            jax.ShapeDtypeStruct((N, M, E), jnp.float32),
            jax.ShapeDtypeStruct((N, M), jnp.float32),
        ],
    )(group_lens, groups, W_nodes, b_nodes2, query, W_query, b_query2,
      probabilities, lens_row)

    return (w.reshape(N, M, 1), out2)


# speedup vs baseline: 1.2896x; 1.1347x over previous
import jax
import jax.numpy as jnp
from jax.experimental import pallas as pl
from jax.experimental.pallas import tpu as pltpu

N, M, E, D = 16, 512, 768, 1024
G = 2
R = G * M
STEPS = N // G


def _main_body(lens_ref, x_ref, w_ref, b_ref, q_in_ref, wq_ref, bq_ref,
               p_ref, lens_v_ref, out2_ref, out1_ref, qt_scratch, dots_s,
               w_bf):
    n = pl.program_id(0)

    @pl.when(n == 0)
    def _():
        qz = jnp.dot(q_in_ref[...].astype(jnp.bfloat16),
                     wq_ref[...].astype(jnp.bfloat16),
                     preferred_element_type=jnp.float32) + bq_ref[...]
        q = jnp.tanh(qz)
        qt_scratch[...] = q.T.astype(jnp.bfloat16)
        w_bf[...] = w_ref[...].astype(jnp.bfloat16)

    L0 = lens_ref[G * n]
    L1 = lens_ref[G * n + 1]

    row_ids = jax.lax.broadcasted_iota(jnp.int32, (R, 1), 0)
    in_first = row_ids < M
    rel_ids = jnp.where(in_first, row_ids, row_ids - M)
    lim = jnp.where(in_first, L0, L1)
    mask = (rel_ids < lim).astype(jnp.float32)
    xm = x_ref[0] * mask
    out2_ref[0] = xm
    z = jnp.dot(xm.astype(jnp.bfloat16), w_bf[...],
                preferred_element_type=jnp.float32) + b_ref[...]
    t = jnp.tanh(z)
    d = jnp.dot(t.astype(jnp.bfloat16), qt_scratch[...],
                preferred_element_type=jnp.float32)

    lane_ids = jax.lax.broadcasted_iota(jnp.int32, (M, N), 1)
    d0 = jnp.broadcast_to(d[:M], (M, N))
    d1 = jnp.broadcast_to(d[M:], (M, N))
    dots_s[...] = jnp.where(lane_ids == G * n, d0,
                            jnp.where(lane_ids == G * n + 1, d1, dots_s[...]))

    @pl.when(n == STEPS - 1)
    def _():
        dd = dots_s[...]
        m1 = jnp.max(dd, axis=0, keepdims=True)
        e1 = jnp.exp(dd - m1)
        a = e1 / jnp.sum(e1, axis=0, keepdims=True)
        logits = a * (p_ref[...] * 10.0)
        g = jnp.max(logits)
        e2 = jnp.exp(logits - g)
        w = e2 / jnp.sum(e2)
        mrow = jax.lax.broadcasted_iota(jnp.int32, (M, 1), 0)
        w = jnp.where(mrow < lens_v_ref[...], w, 0.0)
        out1_ref[...] = w.T


@jax.jit
def kernel(query, groups, probabilities, group_lens, W_nodes, b_nodes,
           W_query, b_query):
    b_nodes2 = b_nodes.reshape(1, D)
    b_query2 = b_query.reshape(1, D)
    lens_row = group_lens.reshape(1, N)
    groups2 = groups.reshape(STEPS, R, E)

    grid_spec = pltpu.PrefetchScalarGridSpec(
        num_scalar_prefetch=1,
        grid=(STEPS,),
        in_specs=[
            pl.BlockSpec((1, R, E), lambda n, lens: (n, 0, 0)),
            pl.BlockSpec((E, D), lambda n, lens: (0, 0)),
            pl.BlockSpec((1, D), lambda n, lens: (0, 0)),
            pl.BlockSpec((1, D), lambda n, lens: (0, 0)),
            pl.BlockSpec((D, D), lambda n, lens: (0, 0)),
            pl.BlockSpec((1, D), lambda n, lens: (0, 0)),
            pl.BlockSpec((1, N), lambda n, lens: (0, 0)),
            pl.BlockSpec((1, N), lambda n, lens: (0, 0)),
        ],
        out_specs=[
            pl.BlockSpec((1, R, E), lambda n, lens: (n, 0, 0)),
            pl.BlockSpec((N, M), lambda n, lens: (0, 0)),
        ],
        scratch_shapes=[
            pltpu.VMEM((D, 1), jnp.bfloat16),
            pltpu.VMEM((M, N), jnp.float32),
            pltpu.VMEM((E, D), jnp.bfloat16),
        ],
    )
    out2, w = pl.pallas_call(
        _main_body,
        grid_spec=grid_spec,
        out_shape=[
            jax.ShapeDtypeStruct((STEPS, R, E), jnp.float32),
            jax.ShapeDtypeStruct((N, M), jnp.float32),
        ],
    )(group_lens, groups2, W_nodes, b_nodes2, query, W_query, b_query2,
      probabilities, lens_row)

    return (w.reshape(N, M, 1), out2.reshape(N, M, E))
